# baseline (speedup 1.0000x reference)
import functools

import jax
import jax.numpy as jnp
from jax import lax
from jax.experimental import pallas as pl
from jax.experimental.pallas import tpu as pltpu
from jax.experimental.pallas import tpu_sc as plsc

N = 10000
F = 128
E = 320000
NC = 2
NS = 16
NW = NC * NS
BLK = 128
B_C0 = 80
B_C1 = 80
BMAX = max(B_C0, B_C1)
TB = NS * (B_C0 + B_C1)
E_PAD = TB * BLK
ACC_ROWS = NS * 640
DUMP_ROW = N
ZROWS = 640
FH = 64
BOUND = 10.0


def _make_sc_agg():
    mesh = plsc.VectorSubcoreMesh(core_axis_name="c", subcore_axis_name="s")
    out_type = (
        jax.ShapeDtypeStruct((NC, 2, ACC_ROWS, FH), jnp.float32),
        jax.ShapeDtypeStruct((NC, 2, ACC_ROWS, FH), jnp.float32),
    )
    scratch = [
        pltpu.VMEM((BMAX, BLK), jnp.int32),
        pltpu.VMEM((BMAX, BLK), jnp.int32),
        pltpu.VMEM((8, BLK, FH), jnp.float32),
        pltpu.VMEM_SHARED((ACC_ROWS, FH), jnp.float32),
        [pltpu.SemaphoreType.DMA] * 8,
    ]

    @functools.partial(
        pl.kernel, mesh=mesh, out_type=out_type, scratch_types=scratch,
        compiler_params=pltpu.CompilerParams(use_tc_tiling_on_sc=False))
    def sc_agg(xl0, xl1, xr0, xr1, sg, ss, dg, ds, zeros, out_r, out_l,
               gidx, sidx, rows, acc, gsem):
        c = lax.axis_index("c")
        s = lax.axis_index("s")

        def ring(x_hbm, nblk):
            def gather(j, b):
                pltpu.async_copy(x_hbm.at[gidx.at[j]], rows.at[b], gsem[b])

            for b in range(8):
                gather(b, b)

            def quad(i, carry):
                for b in range(8):
                    j = 8 * i + b
                    pltpu.make_async_copy(
                        x_hbm.at[gidx.at[j]], rows.at[b], gsem[b]).wait()
                    pltpu.sync_copy(rows.at[b], acc.at[sidx.at[j]], add=True)

                    @pl.when(j + 8 < nblk)
                    def _():
                        gather(j + 8, b)
                return carry

            lax.fori_loop(0, nblk // 8, quad, 0)

        def run_half(x_hbm, out_hbm, h):
            pltpu.sync_copy(zeros, acc.at[pl.ds(s * ZROWS, ZROWS)])
            plsc.subcore_barrier()

            @pl.when(c == 0)
            def _():
                ring(x_hbm, B_C0)

            @pl.when(c == 1)
            def _():
                ring(x_hbm, B_C1)

            plsc.subcore_barrier()
            pltpu.sync_copy(acc.at[pl.ds(s * ZROWS, ZROWS)],
                            out_hbm.at[c, h, pl.ds(s * ZROWS, ZROWS)])
            plsc.subcore_barrier()

        def run_dir(x0_hbm, x1_hbm, g_hbm, s_hbm, out_hbm):
            @pl.when(c == 0)
            def _():
                pltpu.sync_copy(g_hbm.at[pl.ds(s * B_C0, B_C0)],
                                gidx.at[pl.ds(0, B_C0)])
                pltpu.sync_copy(s_hbm.at[pl.ds(s * B_C0, B_C0)],
                                sidx.at[pl.ds(0, B_C0)])

            @pl.when(c == 1)
            def _():
                pltpu.sync_copy(g_hbm.at[pl.ds(NS * B_C0 + s * B_C1, B_C1)],
                                gidx.at[pl.ds(0, B_C1)])
                pltpu.sync_copy(s_hbm.at[pl.ds(NS * B_C0 + s * B_C1, B_C1)],
                                sidx.at[pl.ds(0, B_C1)])

            run_half(x0_hbm, out_hbm, 0)
            run_half(x1_hbm, out_hbm, 1)

        run_dir(xl0, xl1, sg, ds, out_r)
        run_dir(xr0, xr1, dg, ss, out_l)

    return sc_agg


_SC_AGG = _make_sc_agg()


BR = 1000


def _agg_of(p_ref):
    return jnp.concatenate(
        [p_ref[0, 0] + p_ref[1, 0], p_ref[0, 1] + p_ref[1, 1]], axis=1)


def _conv_body(p_ref, x_ref, wrel_ref, wroot_ref, b_ref, o_ref, *, relu):
    agg = _agg_of(p_ref)
    y = jnp.dot(agg, wrel_ref[...], preferred_element_type=jnp.float32)
    y = y + jnp.dot(x_ref[...], wroot_ref[...],
                    preferred_element_type=jnp.float32)
    y = y + b_ref[...]
    if relu:
        y = jnp.maximum(y, 0.0)
    o_ref[...] = y


def _final_body(p_ref, x_ref, wrel_ref, wroot_ref, b_ref, feas_ref, o_ref):
    agg = _agg_of(p_ref)
    z = jnp.dot(agg, wrel_ref[...], preferred_element_type=jnp.float32)
    z = z + jnp.dot(x_ref[...], wroot_ref[...],
                    preferred_element_type=jnp.float32)
    z = z + b_ref[...]
    nrm = jnp.sqrt(jnp.sum(z * z, axis=1, keepdims=True))
    zn = z / jnp.maximum(nrm, 1e-12) * 10.0
    f = feas_ref[...]
    cl = (jnp.abs(f[:, F - 3:F - 2]) != 0.0).astype(jnp.float32)
    cu = (jnp.abs(f[:, F - 1:F]) != 0.0).astype(jnp.float32)
    col = lax.broadcasted_iota(jnp.int32, zn.shape, 1)
    zn = zn + jnp.where(col == 0, -BOUND * cl, 0.0)
    zn = zn + jnp.where(col == 2, -BOUND * cu, 0.0)
    o_ref[...] = zn


def _conv_tc(p, x_old, w_rel, w_root, b, relu):
    grid = (N // BR,)
    return pl.pallas_call(
        functools.partial(_conv_body, relu=relu),
        grid=grid,
        in_specs=[
            pl.BlockSpec((NC, 2, BR, FH), lambda i: (0, 0, i, 0)),
            pl.BlockSpec((BR, F), lambda i: (i, 0)),
            pl.BlockSpec((F, F), lambda i: (0, 0)),
            pl.BlockSpec((F, F), lambda i: (0, 0)),
            pl.BlockSpec((1, F), lambda i: (0, 0)),
        ],
        out_specs=pl.BlockSpec((BR, F), lambda i: (i, 0)),
        out_shape=jax.ShapeDtypeStruct((N, F), jnp.float32),
    )(p, x_old, w_rel, w_root, b)


def _final_tc(p, x_old, w_rel, w_root, b, feas):
    grid = (N // BR,)
    return pl.pallas_call(
        _final_body,
        grid=grid,
        in_specs=[
            pl.BlockSpec((NC, 2, BR, FH), lambda i: (0, 0, i, 0)),
            pl.BlockSpec((BR, F), lambda i: (i, 0)),
            pl.BlockSpec((F, F), lambda i: (0, 0)),
            pl.BlockSpec((F, F), lambda i: (0, 0)),
            pl.BlockSpec((1, F), lambda i: (0, 0)),
            pl.BlockSpec((BR, F), lambda i: (i, 0)),
        ],
        out_specs=pl.BlockSpec((BR, F), lambda i: (i, 0)),
        out_shape=jax.ShapeDtypeStruct((N, F), jnp.float32),
    )(p, x_old, w_rel, w_root, b, feas)


def _pad_w(w):
    return jnp.pad(w, ((0, 0), (0, F - w.shape[1])))


def _pad_b(b):
    return jnp.pad(b, (0, F - b.shape[0])).reshape(1, F)


def kernel(x_s, x_t, edge_index, params):
    src = edge_index[0].astype(jnp.int32)
    dst = edge_index[1].astype(jnp.int32)
    pad = E_PAD - E
    slab = (TB, BLK)
    dump = DUMP_ROW + (jnp.arange(pad, dtype=jnp.int32) % (ACC_ROWS - N))
    gpad = jnp.arange(pad, dtype=jnp.int32) % N
    sg = jnp.concatenate([src, gpad]).reshape(slab)
    ss = jnp.concatenate([src, dump]).reshape(slab)
    dg = jnp.concatenate([dst, gpad]).reshape(slab)
    ds = jnp.concatenate([dst, dump]).reshape(slab)
    zeros = jnp.zeros((ZROWS, FH), jnp.float32)

    def halves(x):
        return x[:, :FH], x[:, FH:]

    l, r = x_s, x_t
    for li in (1, 2):
        pr, plft = _SC_AGG(*halves(l), *halves(r), sg, ss, dg, ds, zeros)
        r_new = _conv_tc(pr, r, params['w%d_l2r_rel' % li],
                         params['w%d_l2r_root' % li],
                         params['b%d_l2r' % li].reshape(1, F), relu=True)
        l_new = _conv_tc(plft, l, params['w%d_r2l_rel' % li],
                         params['w%d_r2l_root' % li],
                         params['b%d_r2l' % li].reshape(1, F), relu=True)
        l, r = l_new, r_new

    pr, plft = _SC_AGG(*halves(l), *halves(r), sg, ss, dg, ds, zeros)
    r_out = _final_tc(pr, r, _pad_w(params['w3_l2r_rel']),
                      _pad_w(params['w3_l2r_root']),
                      _pad_b(params['b3_l2r']), x_t)
    l_out = _final_tc(plft, l, _pad_w(params['w3_r2l_rel']),
                      _pad_w(params['w3_r2l_root']),
                      _pad_b(params['b3_r2l']), x_s)
    return l_out[:, :3], r_out[:, :3]


# speedup vs baseline: 1.0049x; 1.0049x over previous
import functools

import jax
import jax.numpy as jnp
from jax import lax
from jax.experimental import pallas as pl
from jax.experimental.pallas import tpu as pltpu
from jax.experimental.pallas import tpu_sc as plsc

N = 10000
F = 128
E = 320000
NC = 2
NS = 16
NW = NC * NS
BLK = 128
B_C0 = 80
B_C1 = 80
BMAX = max(B_C0, B_C1)
TB = NS * (B_C0 + B_C1)
E_PAD = TB * BLK
ACC_ROWS = NS * 640
DUMP_ROW = N
ZROWS = 640
FH = 64
BOUND = 10.0


def _make_sc_agg():
    mesh = plsc.VectorSubcoreMesh(core_axis_name="c", subcore_axis_name="s")
    out_type = (
        jax.ShapeDtypeStruct((NC, 2, ACC_ROWS, FH), jnp.float32),
        jax.ShapeDtypeStruct((NC, 2, ACC_ROWS, FH), jnp.float32),
    )
    scratch = [
        pltpu.VMEM((BMAX, BLK), jnp.int32),
        pltpu.VMEM((BMAX, BLK), jnp.int32),
        pltpu.VMEM((4, BLK, FH), jnp.float32),
        pltpu.VMEM_SHARED((ACC_ROWS, FH), jnp.float32),
        [pltpu.SemaphoreType.DMA] * 4,
    ]

    @functools.partial(
        pl.kernel, mesh=mesh, out_type=out_type, scratch_types=scratch,
        compiler_params=pltpu.CompilerParams(use_tc_tiling_on_sc=False))
    def sc_agg(xl0, xl1, xr0, xr1, sg, ss, dg, ds, zeros, out_r, out_l,
               gidx, sidx, rows, acc, gsem):
        c = lax.axis_index("c")
        s = lax.axis_index("s")

        def ring(x_hbm, nblk):
            def gather(j, b):
                pltpu.async_copy(x_hbm.at[gidx.at[j]], rows.at[b], gsem[b])

            for b in range(4):
                gather(b, b)

            def quad(i, carry):
                for b in range(4):
                    j = 4 * i + b
                    pltpu.make_async_copy(
                        x_hbm.at[gidx.at[j]], rows.at[b], gsem[b]).wait()
                    pltpu.sync_copy(rows.at[b], acc.at[sidx.at[j]], add=True)

                    @pl.when(j + 4 < nblk)
                    def _():
                        gather(j + 4, b)
                return carry

            lax.fori_loop(0, nblk // 4, quad, 0)

        def run_half(x_hbm, out_hbm, h):
            pltpu.sync_copy(zeros, acc.at[pl.ds(s * ZROWS, ZROWS)])
            plsc.subcore_barrier()

            @pl.when(c == 0)
            def _():
                ring(x_hbm, B_C0)

            @pl.when(c == 1)
            def _():
                ring(x_hbm, B_C1)

            plsc.subcore_barrier()
            pltpu.sync_copy(acc.at[pl.ds(s * ZROWS, ZROWS)],
                            out_hbm.at[c, h, pl.ds(s * ZROWS, ZROWS)])
            plsc.subcore_barrier()

        def run_dir(x0_hbm, x1_hbm, g_hbm, s_hbm, out_hbm):
            @pl.when(c == 0)
            def _():
                pltpu.sync_copy(g_hbm.at[pl.ds(s * B_C0, B_C0)],
                                gidx.at[pl.ds(0, B_C0)])
                pltpu.sync_copy(s_hbm.at[pl.ds(s * B_C0, B_C0)],
                                sidx.at[pl.ds(0, B_C0)])

            @pl.when(c == 1)
            def _():
                pltpu.sync_copy(g_hbm.at[pl.ds(NS * B_C0 + s * B_C1, B_C1)],
                                gidx.at[pl.ds(0, B_C1)])
                pltpu.sync_copy(s_hbm.at[pl.ds(NS * B_C0 + s * B_C1, B_C1)],
                                sidx.at[pl.ds(0, B_C1)])

            run_half(x0_hbm, out_hbm, 0)
            run_half(x1_hbm, out_hbm, 1)

        run_dir(xl0, xl1, sg, ds, out_r)
        run_dir(xr0, xr1, dg, ss, out_l)

    return sc_agg


_SC_AGG = _make_sc_agg()


BR = 1000


def _agg_of(p_ref):
    return jnp.concatenate(
        [p_ref[0, 0] + p_ref[1, 0], p_ref[0, 1] + p_ref[1, 1]], axis=1)


def _conv_body(p_ref, x_ref, wrel_ref, wroot_ref, b_ref, o_ref, *, relu):
    agg = _agg_of(p_ref)
    y = jnp.dot(agg, wrel_ref[...], preferred_element_type=jnp.float32)
    y = y + jnp.dot(x_ref[...], wroot_ref[...],
                    preferred_element_type=jnp.float32)
    y = y + b_ref[...]
    if relu:
        y = jnp.maximum(y, 0.0)
    o_ref[...] = y


def _final_body(p_ref, x_ref, wrel_ref, wroot_ref, b_ref, feas_ref, o_ref):
    agg = _agg_of(p_ref)
    z = jnp.dot(agg, wrel_ref[...], preferred_element_type=jnp.float32)
    z = z + jnp.dot(x_ref[...], wroot_ref[...],
                    preferred_element_type=jnp.float32)
    z = z + b_ref[...]
    nrm = jnp.sqrt(jnp.sum(z * z, axis=1, keepdims=True))
    zn = z / jnp.maximum(nrm, 1e-12) * 10.0
    f = feas_ref[...]
    cl = (jnp.abs(f[:, F - 3:F - 2]) != 0.0).astype(jnp.float32)
    cu = (jnp.abs(f[:, F - 1:F]) != 0.0).astype(jnp.float32)
    col = lax.broadcasted_iota(jnp.int32, zn.shape, 1)
    zn = zn + jnp.where(col == 0, -BOUND * cl, 0.0)
    zn = zn + jnp.where(col == 2, -BOUND * cu, 0.0)
    o_ref[...] = zn


def _conv_tc(p, x_old, w_rel, w_root, b, relu):
    grid = (N // BR,)
    return pl.pallas_call(
        functools.partial(_conv_body, relu=relu),
        grid=grid,
        in_specs=[
            pl.BlockSpec((NC, 2, BR, FH), lambda i: (0, 0, i, 0)),
            pl.BlockSpec((BR, F), lambda i: (i, 0)),
            pl.BlockSpec((F, F), lambda i: (0, 0)),
            pl.BlockSpec((F, F), lambda i: (0, 0)),
            pl.BlockSpec((1, F), lambda i: (0, 0)),
        ],
        out_specs=pl.BlockSpec((BR, F), lambda i: (i, 0)),
        out_shape=jax.ShapeDtypeStruct((N, F), jnp.float32),
    )(p, x_old, w_rel, w_root, b)


def _final_tc(p, x_old, w_rel, w_root, b, feas):
    grid = (N // BR,)
    return pl.pallas_call(
        _final_body,
        grid=grid,
        in_specs=[
            pl.BlockSpec((NC, 2, BR, FH), lambda i: (0, 0, i, 0)),
            pl.BlockSpec((BR, F), lambda i: (i, 0)),
            pl.BlockSpec((F, F), lambda i: (0, 0)),
            pl.BlockSpec((F, F), lambda i: (0, 0)),
            pl.BlockSpec((1, F), lambda i: (0, 0)),
            pl.BlockSpec((BR, F), lambda i: (i, 0)),
        ],
        out_specs=pl.BlockSpec((BR, F), lambda i: (i, 0)),
        out_shape=jax.ShapeDtypeStruct((N, F), jnp.float32),
    )(p, x_old, w_rel, w_root, b, feas)


def _pad_w(w):
    return jnp.pad(w, ((0, 0), (0, F - w.shape[1])))


def _pad_b(b):
    return jnp.pad(b, (0, F - b.shape[0])).reshape(1, F)


def kernel(x_s, x_t, edge_index, params):
    src = edge_index[0].astype(jnp.int32)
    dst = edge_index[1].astype(jnp.int32)
    pad = E_PAD - E
    slab = (TB, BLK)
    dump = DUMP_ROW + (jnp.arange(pad, dtype=jnp.int32) % (ACC_ROWS - N))
    gpad = jnp.arange(pad, dtype=jnp.int32) % N
    sg = jnp.concatenate([src, gpad]).reshape(slab)
    ss = jnp.concatenate([src, dump]).reshape(slab)
    dg = jnp.concatenate([dst, gpad]).reshape(slab)
    ds = jnp.concatenate([dst, dump]).reshape(slab)
    zeros = jnp.zeros((ZROWS, FH), jnp.float32)

    def halves(x):
        return x[:, :FH], x[:, FH:]

    l, r = x_s, x_t
    for li in (1, 2):
        pr, plft = _SC_AGG(*halves(l), *halves(r), sg, ss, dg, ds, zeros)
        r_new = _conv_tc(pr, r, params['w%d_l2r_rel' % li],
                         params['w%d_l2r_root' % li],
                         params['b%d_l2r' % li].reshape(1, F), relu=True)
        l_new = _conv_tc(plft, l, params['w%d_r2l_rel' % li],
                         params['w%d_r2l_root' % li],
                         params['b%d_r2l' % li].reshape(1, F), relu=True)
        l, r = l_new, r_new

    pr, plft = _SC_AGG(*halves(l), *halves(r), sg, ss, dg, ds, zeros)
    r_out = _final_tc(pr, r, _pad_w(params['w3_l2r_rel']),
                      _pad_w(params['w3_l2r_root']),
                      _pad_b(params['b3_l2r']), x_t)
    l_out = _final_tc(plft, l, _pad_w(params['w3_r2l_rel']),
                      _pad_w(params['w3_r2l_root']),
                      _pad_b(params['b3_r2l']), x_s)
    return l_out[:, :3], r_out[:, :3]
